# Initial kernel scaffold; baseline (speedup 1.0000x reference)
import jax
import jax.numpy as jnp
from jax.experimental import pallas as pl


def kernel(x, edge_index, batch, W1, b1, W2, b2, W3, b3, W4, b4, W5, b5, Wfc, bfc):
    raise NotImplementedError("write your pallas kernel here")


# speedup vs baseline: 7.0064x; 7.0064x over previous
import jax
import jax.numpy as jnp
from jax import lax
from jax.experimental import pallas as pl
from jax.experimental.pallas import tpu as pltpu
from jax.experimental.pallas import tpu_sc as plsc

N = 10000
E = 320000
DIN = 128
H = 256
C = 10
G = 64

NC = 2
NS = 16
CHUNK = 128
TPC = 160
IB = 32
NB = TPC // IB
EPT = TPC * CHUNK
EPAD = NS * EPT
ACC_ROWS = 10240
ZROWS = ACC_ROWS // NS
WROWS = 624
HALF = 128
RB = 1000
GRID = N // RB


def _agg_body(p_hbm, src_hbm, dst_hbm, zeros_hbm, out_hbm,
              src_v, dst_v, rows0, rows1, acc, sem0, sem1):
    c = lax.axis_index("c")
    s = lax.axis_index("s")
    pltpu.sync_copy(zeros_hbm, rows0)
    for k in range(ZROWS // CHUNK):
        pltpu.sync_copy(rows0, acc.at[pl.ds(s * ZROWS + k * CHUNK, CHUNK)])
    plsc.subcore_barrier()
    pslab = p_hbm.at[c]
    sidx = src_hbm.at[s]
    didx = dst_hbm.at[s]
    for blk in range(NB):
        pltpu.sync_copy(sidx.at[pl.ds(blk * IB, IB)], src_v)
        pltpu.sync_copy(didx.at[pl.ds(blk * IB, IB)], dst_v)
        pltpu.make_async_copy(pslab.at[src_v.at[0]], rows0, sem0).start()
        pltpu.make_async_copy(pslab.at[src_v.at[1]], rows1, sem1).start()

        def step(i, carry):
            j0 = 2 * i
            pltpu.make_async_copy(pslab.at[src_v.at[j0]], rows0, sem0).wait()
            pltpu.sync_copy(rows0, acc.at[dst_v.at[j0]], add=True)
            pltpu.make_async_copy(pslab.at[src_v.at[j0 + 2]], rows0, sem0).start()
            pltpu.make_async_copy(pslab.at[src_v.at[j0 + 1]], rows1, sem1).wait()
            pltpu.sync_copy(rows1, acc.at[dst_v.at[j0 + 1]], add=True)
            pltpu.make_async_copy(pslab.at[src_v.at[j0 + 3]], rows1, sem1).start()
            return carry

        lax.fori_loop(0, IB // 2 - 1, step, 0)
        pltpu.make_async_copy(pslab.at[src_v.at[IB - 2]], rows0, sem0).wait()
        pltpu.sync_copy(rows0, acc.at[dst_v.at[IB - 2]], add=True)
        pltpu.make_async_copy(pslab.at[src_v.at[IB - 1]], rows1, sem1).wait()
        pltpu.sync_copy(rows1, acc.at[dst_v.at[IB - 1]], add=True)
    plsc.subcore_barrier()
    r0 = s * WROWS

    @pl.when(s < NS - 1)
    def _():
        pltpu.sync_copy(acc.at[pl.ds(r0, WROWS)],
                        out_hbm.at[c].at[pl.ds(r0, WROWS)])

    @pl.when(s == NS - 1)
    def _():
        pltpu.sync_copy(acc.at[pl.ds((NS - 1) * WROWS, N - (NS - 1) * WROWS)],
                        out_hbm.at[c].at[pl.ds((NS - 1) * WROWS,
                                               N - (NS - 1) * WROWS)])


_agg_call = pl.kernel(
    _agg_body,
    jax.ShapeDtypeStruct((NC, N, HALF), jnp.float32),
    mesh=plsc.VectorSubcoreMesh(core_axis_name="c", subcore_axis_name="s"),
    scratch_types=[
        pltpu.VMEM((IB, CHUNK), jnp.int32),
        pltpu.VMEM((IB, CHUNK), jnp.int32),
        pltpu.VMEM((CHUNK, HALF), jnp.float32),
        pltpu.VMEM((CHUNK, HALF), jnp.float32),
        pltpu.VMEM_SHARED((ACC_ROWS, HALF), jnp.float32),
        pltpu.SemaphoreType.DMA,
        pltpu.SemaphoreType.DMA,
    ],
)


def _deg_body(dst_hbm, ones_hbm, zeros_hbm, out_hbm, dst_v, ones_v, accd):
    c = lax.axis_index("c")
    s = lax.axis_index("s")
    pltpu.sync_copy(dst_hbm.at[s], dst_v)
    pltpu.sync_copy(ones_hbm, ones_v)
    pltpu.sync_copy(zeros_hbm, accd.at[pl.ds(s * ZROWS, ZROWS)])
    plsc.subcore_barrier()
    half = TPC // 2

    def step(i, carry):
        j = c * half + i
        pltpu.sync_copy(ones_v, accd.at[dst_v.at[j]], add=True)
        return carry

    lax.fori_loop(0, half, step, 0)
    plsc.subcore_barrier()
    r0 = s * ZROWS
    pltpu.sync_copy(accd.at[pl.ds(r0, ZROWS)],
                    out_hbm.at[c].at[pl.ds(r0, ZROWS)])


_deg_call = pl.kernel(
    _deg_body,
    jax.ShapeDtypeStruct((NC, ACC_ROWS, 8), jnp.float32),
    mesh=plsc.VectorSubcoreMesh(core_axis_name="c", subcore_axis_name="s"),
    scratch_types=[
        pltpu.VMEM((TPC, CHUNK), jnp.int32),
        pltpu.VMEM((CHUNK, 8), jnp.float32),
        pltpu.VMEM_SHARED((ACC_ROWS, 8), jnp.float32),
    ],
)


def _tc_first_body(x_ref, w_ref, degp_ref, p_ref, dis_ref):
    deg = degp_ref[0, :, 0:1] + degp_ref[1, :, 0:1] + 1.0
    dis = lax.rsqrt(deg)
    dis_ref[...] = dis
    u = jnp.dot(x_ref[...], w_ref[...], preferred_element_type=jnp.float32) * dis
    p_ref[0] = u[:, :HALF]
    p_ref[1] = u[:, HALF:]


_tc_first = pl.pallas_call(
    _tc_first_body,
    grid=(GRID,),
    in_specs=[
        pl.BlockSpec((RB, DIN), lambda i: (i, 0)),
        pl.BlockSpec((DIN, H), lambda i: (0, 0)),
        pl.BlockSpec((NC, RB, 8), lambda i: (0, i, 0)),
    ],
    out_specs=[
        pl.BlockSpec((NC, RB, HALF), lambda i: (0, i, 0)),
        pl.BlockSpec((RB, 1), lambda i: (i, 0)),
    ],
    out_shape=[
        jax.ShapeDtypeStruct((NC, N, HALF), jnp.float32),
        jax.ShapeDtypeStruct((N, 1), jnp.float32),
    ],
)


def _tc_mid_body(agg_ref, p_ref, dis_ref, b_ref, w_ref, o_ref):
    dis = dis_ref[...]
    h = jnp.concatenate([agg_ref[0] + p_ref[0], agg_ref[1] + p_ref[1]], axis=1)
    h = jnp.maximum(h * dis + b_ref[...], 0.0)
    u = jnp.dot(h, w_ref[...], preferred_element_type=jnp.float32) * dis
    o_ref[0] = u[:, :HALF]
    o_ref[1] = u[:, HALF:]


_tc_mid = pl.pallas_call(
    _tc_mid_body,
    grid=(GRID,),
    in_specs=[
        pl.BlockSpec((NC, RB, HALF), lambda i: (0, i, 0)),
        pl.BlockSpec((NC, RB, HALF), lambda i: (0, i, 0)),
        pl.BlockSpec((RB, 1), lambda i: (i, 0)),
        pl.BlockSpec((1, H), lambda i: (0, 0)),
        pl.BlockSpec((H, H), lambda i: (0, 0)),
    ],
    out_specs=pl.BlockSpec((NC, RB, HALF), lambda i: (0, i, 0)),
    out_shape=jax.ShapeDtypeStruct((NC, N, HALF), jnp.float32),
)


def _tc_final_body(agg_ref, p_ref, dis_ref, b_ref, batch_ref, wfc_ref, bfc_ref,
                   o_ref, sums, cnt):
    i = pl.program_id(0)

    @pl.when(i == 0)
    def _():
        sums[...] = jnp.zeros_like(sums)
        cnt[...] = jnp.zeros_like(cnt)

    dis = dis_ref[...]
    h = jnp.concatenate([agg_ref[0] + p_ref[0], agg_ref[1] + p_ref[1]], axis=1)
    h = jnp.maximum(h * dis + b_ref[...], 0.0)
    seg = lax.broadcasted_iota(jnp.int32, (G, RB), 0)
    mask = (batch_ref[0] == seg).astype(jnp.float32)
    sums[...] += jnp.dot(mask, h, preferred_element_type=jnp.float32)
    cnt[...] += jnp.sum(mask, axis=1, keepdims=True)

    @pl.when(i == GRID - 1)
    def _():
        pooled = sums[...] / jnp.maximum(cnt[...], 1.0)
        o_ref[...] = (jnp.dot(pooled, wfc_ref[...],
                              preferred_element_type=jnp.float32) + bfc_ref[...])


_tc_final = pl.pallas_call(
    _tc_final_body,
    grid=(GRID,),
    in_specs=[
        pl.BlockSpec((NC, RB, HALF), lambda i: (0, i, 0)),
        pl.BlockSpec((NC, RB, HALF), lambda i: (0, i, 0)),
        pl.BlockSpec((RB, 1), lambda i: (i, 0)),
        pl.BlockSpec((1, H), lambda i: (0, 0)),
        pl.BlockSpec((1, 1, RB), lambda i: (i, 0, 0)),
        pl.BlockSpec((H, C), lambda i: (0, 0)),
        pl.BlockSpec((1, C), lambda i: (0, 0)),
    ],
    out_specs=pl.BlockSpec((G, C), lambda i: (0, 0)),
    out_shape=jax.ShapeDtypeStruct((G, C), jnp.float32),
    scratch_shapes=[
        pltpu.VMEM((G, H), jnp.float32),
        pltpu.VMEM((G, 1), jnp.float32),
    ],
)


def kernel(x, edge_index, batch, W1, b1, W2, b2, W3, b3, W4, b4, W5, b5, Wfc, bfc):
    src = edge_index[0]
    dst = edge_index[1]
    pad = EPAD - E
    psrc = jnp.concatenate([src, jnp.zeros((pad,), jnp.int32)])
    pdst = jnp.concatenate(
        [dst, N + (jnp.arange(pad, dtype=jnp.int32) % (ACC_ROWS - N))])
    src_r = psrc.reshape(NS, TPC, CHUNK)
    dst_r = pdst.reshape(NS, TPC, CHUNK)
    zeros128 = jnp.zeros((CHUNK, HALF), jnp.float32)
    ones8 = jnp.ones((CHUNK, 8), jnp.float32)
    zeros8 = jnp.zeros((ZROWS, 8), jnp.float32)

    degp = _deg_call(dst_r, ones8, zeros8)
    p, dis = _tc_first(x, W1, degp)
    Ws = [W2, W3, W4, W5]
    bs = [b1, b2, b3, b4]
    for l in range(4):
        agg = _agg_call(p, src_r, dst_r, zeros128)
        p = _tc_mid(agg, p, dis, bs[l].reshape(1, H), Ws[l])
    agg = _agg_call(p, src_r, dst_r, zeros128)
    return _tc_final(agg, p, dis, b5.reshape(1, H), batch.reshape(GRID, 1, RB),
                     Wfc, bfc.reshape(1, C))
